# baseline (speedup 1.0000x reference)
import functools

import jax
import jax.numpy as jnp
from jax import lax
from jax.experimental import pallas as pl
from jax.experimental.pallas import tpu as pltpu
from jax.experimental.pallas import tpu_sc as plsc

_GRP = 8


@functools.lru_cache(maxsize=None)
def _make_gather(V, D, B):
  info = plsc.get_sparse_core_info()
  NC, NS, L = info.num_cores, info.num_subcores, info.num_lanes
  NW = NC * NS
  assert B % NW == 0 and D % 8 == 0 and D % L == 0
  b_per_w = B // NW
  n_groups = b_per_w // _GRP
  mesh = plsc.VectorSubcoreMesh(core_axis_name="c", subcore_axis_name="s")

  @functools.partial(
      pl.kernel,
      mesh=mesh,
      compiler_params=pltpu.CompilerParams(needs_layout_passes=False),
      out_type=jax.ShapeDtypeStruct((D, B), jnp.float32),
      scratch_types=[
          pltpu.VMEM((b_per_w + _GRP,), jnp.int32),
          pltpu.VMEM((3, _GRP, D, 128), jnp.float32),
          pltpu.VMEM((D, b_per_w), jnp.float32),
          pltpu.SemaphoreType.DMA,
      ],
  )
  def k(tblT_hbm, idx_hbm, outT_hbm, idx_v, bufs, slab, sem):
    wid = lax.axis_index("s") * NC + lax.axis_index("c")
    base = wid * b_per_w
    pltpu.sync_copy(
        idx_hbm.at[pl.ds(base, b_per_w)], idx_v.at[pl.ds(0, b_per_w)]
    )

    def group_vec(g):
      return idx_v[pl.ds(g * _GRP, L)]

    def issue(bank, g):
      cv = (group_vec(g) >> 7) << 7
      for b in range(_GRP):
        c0 = pl.multiple_of(cv[b], 128)
        pltpu.async_copy(
            tblT_hbm.at[:, pl.ds(c0, 128)], bufs.at[bank, b], sem
        )

    def drain_and_extract(bank, g):
      rows0 = lax.iota(jnp.int32, L)
      lv = group_vec(g) & 127
      for b in range(_GRP):
        pltpu.make_async_copy(
            tblT_hbm.at[:, pl.ds(0, 128)], bufs.at[bank, b], sem
        ).wait()
        lane = jnp.full((L,), lv[b], jnp.int32)
        col = jnp.full((L,), g * _GRP + b, jnp.int32)
        win = bufs.at[bank, b]
        for h in range(D // L):
          rows = rows0 + h * L
          v = plsc.load_gather(win, [rows, lane])
          plsc.store_scatter(slab, [rows, col], v)

    assert (n_groups - 1) % 3 == 0
    issue(0, 0)
    issue(1, 1)

    def body(m, _):
      g0 = m * 3
      issue(2, g0 + 2)
      drain_and_extract(0, g0)
      issue(0, g0 + 3)
      drain_and_extract(1, g0 + 1)

      @pl.when(g0 + 4 < n_groups)
      def _():
        issue(1, g0 + 4)

      drain_and_extract(2, g0 + 2)
      return 0

    lax.fori_loop(0, (n_groups - 1) // 3, body, 0)
    drain_and_extract(0, n_groups - 1)
    pltpu.sync_copy(slab, outT_hbm.at[:, pl.ds(base, b_per_w)])

  return k


def kernel(nodes, table):
  B = nodes.shape[0]
  V, D = table.shape
  f = _make_gather(V, D, B)
  outT = f(table.T, nodes.astype(jnp.int32))
  return outT.T


# speedup vs baseline: 1.0265x; 1.0265x over previous
import functools

import jax
import jax.numpy as jnp
from jax import lax
from jax.experimental import pallas as pl
from jax.experimental.pallas import tpu as pltpu
from jax.experimental.pallas import tpu_sc as plsc

_GRP = 8


@functools.lru_cache(maxsize=None)
def _make_gather(V, D, B):
  info = plsc.get_sparse_core_info()
  NC, NS, L = info.num_cores, info.num_subcores, info.num_lanes
  NW = NC * NS
  assert B % NW == 0 and D % 8 == 0 and D % L == 0
  b_per_w = B // NW
  n_groups = b_per_w // _GRP
  assert n_groups % 2 == 0
  mesh = plsc.VectorSubcoreMesh(core_axis_name="c", subcore_axis_name="s")

  @functools.partial(
      pl.kernel,
      mesh=mesh,
      compiler_params=pltpu.CompilerParams(needs_layout_passes=False),
      out_type=jax.ShapeDtypeStruct((D, B), jnp.float32),
      scratch_types=[
          pltpu.VMEM((b_per_w + _GRP,), jnp.int32),
          pltpu.VMEM((2, _GRP, D, 128), jnp.float32),
          pltpu.VMEM((D, b_per_w), jnp.float32),
          pltpu.SemaphoreType.DMA,
      ],
  )
  def k(tblT_hbm, idx_hbm, outT_hbm, idx_v, bufs, slab, sem):
    wid = lax.axis_index("s") * NC + lax.axis_index("c")
    base = wid * b_per_w
    pltpu.sync_copy(
        idx_hbm.at[pl.ds(base, b_per_w)], idx_v.at[pl.ds(0, b_per_w)]
    )

    def group_vec(g):
      return idx_v[pl.ds(g * _GRP, L)]

    def issue(bank, g):
      cv = (group_vec(g) >> 7) << 7
      for b in range(_GRP):
        c0 = pl.multiple_of(cv[b], 128)
        pltpu.async_copy(
            tblT_hbm.at[:, pl.ds(c0, 128)], bufs.at[bank, b], sem
        )

    def drain_and_extract(bank, g):
      rows0 = lax.iota(jnp.int32, L)
      lv = group_vec(g) & 127
      for b in range(_GRP):
        pltpu.make_async_copy(
            tblT_hbm.at[:, pl.ds(0, 128)], bufs.at[bank, b], sem
        ).wait()
      for b in range(_GRP):
        lane = jnp.full((L,), lv[b], jnp.int32)
        col = jnp.full((L,), g * _GRP + b, jnp.int32)
        win = bufs.at[bank, b]
        for h in range(D // L):
          rows = rows0 + h * L
          v = plsc.load_gather(win, [rows, lane])
          plsc.store_scatter(slab, [rows, col], v)

    issue(0, 0)

    def body(k2, _):
      g0 = k2 * 2
      issue(1, g0 + 1)
      drain_and_extract(0, g0)

      @pl.when(g0 + 2 < n_groups)
      def _():
        issue(0, g0 + 2)

      drain_and_extract(1, g0 + 1)
      return 0

    lax.fori_loop(0, n_groups // 2, body, 0)
    pltpu.sync_copy(slab, outT_hbm.at[:, pl.ds(base, b_per_w)])

  return k


def kernel(nodes, table):
  B = nodes.shape[0]
  V, D = table.shape
  f = _make_gather(V, D, B)
  outT = f(table.T, nodes.astype(jnp.int32))
  return outT.T
